# baseline (speedup 1.0000x reference)
import functools

import jax
import jax.numpy as jnp
from jax import lax
from jax.experimental import pallas as pl
from jax.experimental.pallas import tpu as pltpu
from jax.experimental.pallas import tpu_sc as plsc

EMBED_DIM = 64
SEQ_LEN = 200
NUM_CORES = 2
NUM_SUBCORES = 16
NUM_WORKERS = NUM_CORES * NUM_SUBCORES
CHUNK = 40
VREGS_PER_ROW = EMBED_DIM // 16


def _sc_body(n_chunks, x_hbm, table_hbm, pe_hbm, out_hbm,
             idx_v, pe_v, rows_v, gsem):
    wid = lax.axis_index("s") * NUM_CORES + lax.axis_index("c")
    rows_per_w = n_chunks * CHUNK
    base = wid * n_chunks

    pltpu.sync_copy(x_hbm.at[pl.ds(base, n_chunks)], idx_v)
    pltpu.sync_copy(pe_hbm.at[pl.ds(0, SEQ_LEN)], pe_v)

    def chunk_step(j, carry):
        cp = pltpu.async_copy(table_hbm.at[idx_v.at[j]], rows_v, gsem)
        cp.wait()
        part = lax.rem(j, SEQ_LEN // CHUNK) * CHUNK

        def add_row(i, carry2):
            p = part + i
            for q in range(VREGS_PER_ROW):
                sl = pl.ds(q * 16, 16)
                rows_v[i, sl] = rows_v[i, sl] + pe_v[p, sl]
            return carry2

        lax.fori_loop(0, CHUNK, add_row, 0, unroll=2)

        pltpu.sync_copy(
            rows_v, out_hbm.at[pl.ds(wid * rows_per_w + j * CHUNK, CHUNK)])
        return carry

    lax.fori_loop(0, n_chunks, chunk_step, 0)


def kernel(x, token_table, pe_table):
    B, L = x.shape
    total_rows = B * L
    rows_per_w = total_rows // NUM_WORKERS
    n_chunks = rows_per_w // CHUNK
    assert rows_per_w * NUM_WORKERS == total_rows
    assert n_chunks * CHUNK == rows_per_w
    assert L == SEQ_LEN and L % CHUNK == 0

    x_flat = x.reshape(total_rows // CHUNK, CHUNK).astype(jnp.int32)

    mesh = plsc.VectorSubcoreMesh(core_axis_name="c", subcore_axis_name="s")
    run = pl.kernel(
        functools.partial(_sc_body, n_chunks),
        out_type=jax.ShapeDtypeStruct((total_rows, EMBED_DIM), jnp.float32),
        mesh=mesh,
        compiler_params=pltpu.CompilerParams(use_tc_tiling_on_sc=False),
        scratch_types=[
            pltpu.VMEM((n_chunks, CHUNK), jnp.int32),
            pltpu.VMEM((SEQ_LEN, EMBED_DIM), jnp.float32),
            pltpu.VMEM((CHUNK, EMBED_DIM), jnp.float32),
            pltpu.SemaphoreType.DMA,
        ],
    )
    out = run(x_flat, token_table, pe_table)
    return out.reshape(B, L, EMBED_DIM)


# speedup vs baseline: 1.1546x; 1.1546x over previous
import functools

import jax
import jax.numpy as jnp
from jax import lax
from jax.experimental import pallas as pl
from jax.experimental.pallas import tpu as pltpu
from jax.experimental.pallas import tpu_sc as plsc

EMBED_DIM = 64
SEQ_LEN = 200
NUM_CORES = 2
NUM_SUBCORES = 16
NUM_WORKERS = NUM_CORES * NUM_SUBCORES
CHUNK = 40
NBUF = 8
LEAD = 4
VREGS_PER_ROW = EMBED_DIM // 16


def _sc_body(n_chunks, x_hbm, table_hbm, pe_hbm, out_hbm,
             idx_v, pe_v, rows_v, gsems, wsems):
    wid = lax.axis_index("s") * NUM_CORES + lax.axis_index("c")
    rows_per_w = n_chunks * CHUNK
    base = wid * n_chunks
    out_base = wid * rows_per_w

    pltpu.sync_copy(x_hbm.at[pl.ds(base, n_chunks)], idx_v)
    pltpu.sync_copy(pe_hbm.at[pl.ds(0, SEQ_LEN)], pe_v)

    def start_gather(j, b):
        return pltpu.async_copy(
            table_hbm.at[idx_v.at[j]], rows_v.at[b], gsems.at[b])

    def wait_gather(j, b):
        pltpu.make_async_copy(
            table_hbm.at[idx_v.at[j]], rows_v.at[b], gsems.at[b]).wait()

    def start_write(j, b):
        return pltpu.async_copy(
            rows_v.at[b], out_hbm.at[pl.ds(out_base + j * CHUNK, CHUNK)],
            wsems.at[b])

    def wait_write(j, b):
        pltpu.make_async_copy(
            rows_v.at[b], out_hbm.at[pl.ds(out_base + j * CHUNK, CHUNK)],
            wsems.at[b]).wait()

    for j in range(LEAD):
        start_gather(j, j % NBUF)

    def outer(j8, carry):
        for b in range(NBUF):
            j = j8 * NBUF + b
            jn = j + LEAD
            bn = jn % NBUF

            @pl.when(jn < n_chunks)
            def _():
                @pl.when(j >= LEAD)
                def _():
                    wait_write(j - LEAD, bn)
                start_gather(jn, bn)

            wait_gather(j, b)
            part = lax.rem(j, SEQ_LEN // CHUNK) * CHUNK

            def add_row(i, carry2):
                p = part + i
                for q in range(VREGS_PER_ROW):
                    sl = pl.ds(q * 16, 16)
                    rows_v[b, i, sl] = rows_v[b, i, sl] + pe_v[p, sl]
                return carry2

            lax.fori_loop(0, CHUNK, add_row, 0, unroll=4)
            start_write(j, b)
        return carry

    lax.fori_loop(0, n_chunks // NBUF, outer, 0)

    for jj in range(n_chunks - NBUF, n_chunks):
        wait_write(jj, jj % NBUF)


def kernel(x, token_table, pe_table):
    B, L = x.shape
    total_rows = B * L
    rows_per_w = total_rows // NUM_WORKERS
    n_chunks = rows_per_w // CHUNK
    assert rows_per_w * NUM_WORKERS == total_rows
    assert n_chunks * CHUNK == rows_per_w
    assert n_chunks % NBUF == 0
    assert L == SEQ_LEN and L % CHUNK == 0

    x_flat = x.reshape(total_rows // CHUNK, CHUNK).astype(jnp.int32)

    mesh = plsc.VectorSubcoreMesh(core_axis_name="c", subcore_axis_name="s")
    run = pl.kernel(
        functools.partial(_sc_body, n_chunks),
        out_type=jax.ShapeDtypeStruct((total_rows, EMBED_DIM), jnp.float32),
        mesh=mesh,
        compiler_params=pltpu.CompilerParams(use_tc_tiling_on_sc=False),
        scratch_types=[
            pltpu.VMEM((n_chunks, CHUNK), jnp.int32),
            pltpu.VMEM((SEQ_LEN, EMBED_DIM), jnp.float32),
            pltpu.VMEM((NBUF, CHUNK, EMBED_DIM), jnp.float32),
            pltpu.SemaphoreType.DMA((NBUF,)),
            pltpu.SemaphoreType.DMA((NBUF,)),
        ],
    )
    out = run(x_flat, token_table, pe_table)
    return out.reshape(B, L, EMBED_DIM)


# speedup vs baseline: 1.3446x; 1.1646x over previous
import functools

import jax
import jax.numpy as jnp
from jax import lax
from jax.experimental import pallas as pl
from jax.experimental.pallas import tpu as pltpu
from jax.experimental.pallas import tpu_sc as plsc

EMBED_DIM = 64
SEQ_LEN = 200
BATCH = 1024
NUM_CORES = 2
NUM_SUBCORES = 16
NUM_WORKERS = NUM_CORES * NUM_SUBCORES
CHUNK = 128
TC_PER_L = BATCH // CHUNK
NBUF = 5
LEAD = 2
LANES = 16


def _sc_body(n_chunks, x_hbm, table_hbm, pe_hbm, out_hbm,
             idx_v, pe_v, rows_v, gsems, wsems):
    wid = lax.axis_index("s") * NUM_CORES + lax.axis_index("c")
    ids_per_w = n_chunks * CHUNK
    ci0 = wid * n_chunks

    pltpu.sync_copy(x_hbm.at[pl.ds(wid * ids_per_w, ids_per_w)], idx_v)
    pltpu.sync_copy(pe_hbm.at[pl.ds(0, SEQ_LEN)], pe_v)

    def start_gather(j, b):
        pltpu.async_copy(
            table_hbm.at[idx_v.at[pl.ds(j * CHUNK, CHUNK)]],
            rows_v.at[b], gsems.at[b])

    def wait_gather(j, b):
        pltpu.make_async_copy(
            table_hbm.at[idx_v.at[pl.ds(j * CHUNK, CHUNK)]],
            rows_v.at[b], gsems.at[b]).wait()

    def start_write(j, b):
        ci = ci0 + j
        l = ci // TC_PER_L
        tc = lax.rem(ci, TC_PER_L)
        pltpu.async_copy(
            rows_v.at[b],
            out_hbm.at[pl.ds(tc * CHUNK, CHUNK),
                       pl.ds(l * EMBED_DIM, EMBED_DIM)],
            wsems.at[b])

    def wait_write(b):
        pltpu.make_async_copy(
            rows_v.at[b],
            out_hbm.at[pl.ds(0, CHUNK), pl.ds(0, EMBED_DIM)],
            wsems.at[b]).wait()

    def compute(j, b):
        ci = ci0 + j
        l = ci // TC_PER_L
        pe_q = [pe_v[l, pl.ds(q * LANES, LANES)]
                for q in range(EMBED_DIM // LANES)]

        def add_row(i, carry2):
            for q in range(EMBED_DIM // LANES):
                sl = pl.ds(q * LANES, LANES)
                rows_v[b, i, sl] = rows_v[b, i, sl] + pe_q[q]
            return carry2

        lax.fori_loop(0, CHUNK, add_row, 0, unroll=4)

    for j in range(LEAD):
        start_gather(j, j % NBUF)

    def outer(j5, carry):
        for b in range(NBUF):
            j = j5 * NBUF + b
            bn = (b + LEAD) % NBUF

            @pl.when(j + LEAD < n_chunks)
            def _():
                @pl.when(j >= NBUF - LEAD)
                def _():
                    wait_write(bn)
                start_gather(j + LEAD, bn)

            wait_gather(j, b)
            compute(j, b)
            start_write(j, b)
        return carry

    lax.fori_loop(0, n_chunks // NBUF, outer, 0)

    for b in range(NBUF):
        wait_write(b)


def kernel(x, token_table, pe_table):
    B, L = x.shape
    total = B * L
    n_chunks = total // (NUM_WORKERS * CHUNK)
    assert n_chunks * NUM_WORKERS * CHUNK == total
    assert n_chunks % NBUF == 0 and NBUF > LEAD
    assert B % CHUNK == 0 and L == SEQ_LEN

    x_flat = x.T.reshape(total).astype(jnp.int32)
    pe_s = pe_table[:SEQ_LEN]

    mesh = plsc.VectorSubcoreMesh(core_axis_name="c", subcore_axis_name="s")
    run = pl.kernel(
        functools.partial(_sc_body, n_chunks),
        out_type=jax.ShapeDtypeStruct((BATCH, SEQ_LEN * EMBED_DIM),
                                      jnp.float32),
        mesh=mesh,
        compiler_params=pltpu.CompilerParams(use_tc_tiling_on_sc=False),
        scratch_types=[
            pltpu.VMEM((n_chunks * CHUNK,), jnp.int32),
            pltpu.VMEM((SEQ_LEN, EMBED_DIM), jnp.float32),
            pltpu.VMEM((NBUF, CHUNK, EMBED_DIM), jnp.float32),
            pltpu.SemaphoreType.DMA((NBUF,)),
            pltpu.SemaphoreType.DMA((NBUF,)),
        ],
    )
    out = run(x_flat, token_table, pe_s)
    return out.reshape(B, L, EMBED_DIM)
